# Initial kernel scaffold; baseline (speedup 1.0000x reference)
import jax
import jax.numpy as jnp
from jax.experimental import pallas as pl


def kernel(x, edge_index, Wl0, bl0, Wr0, g0, b0, Wl1, bl1, Wr1, g1, b1, Wl2, bl2, Wr2):
    raise NotImplementedError("write your pallas kernel here")


# speedup vs baseline: 7.6782x; 7.6782x over previous
import jax
import jax.numpy as jnp
from jax import lax
from jax.experimental import pallas as pl
from jax.experimental.pallas import tpu as pltpu
from jax.experimental.pallas import tpu_sc as plsc

N = 10000
NPAD = 10240
E = 320000
EPAD = 327680
D = 128
H2 = 256
EPS = 1e-5

NC, NS, L = 2, 16, 16
SUB = 128
NSUB = 1
CHUNK = SUB * NSUB
ROWS_PER_TILE = NPAD // NS
ZROWS = 64

B = 1280
G = NPAD // B


def _make_sc_agg(feat_split: bool, with_deg: bool):
  mesh = plsc.VectorSubcoreMesh(core_axis_name="c", subcore_axis_name="s",
                                num_cores=NC, num_subcores=NS)
  out_type = [jax.ShapeDtypeStruct((2, NPAD, D), jnp.float32)]
  scratch = [
      pltpu.VMEM_SHARED((NPAD, D), jnp.float32),
      pltpu.VMEM((CHUNK, D), jnp.float32),
      pltpu.VMEM((CHUNK, D), jnp.float32),
      pltpu.VMEM((CHUNK,), jnp.int32),
      pltpu.VMEM((CHUNK,), jnp.int32),
      pltpu.VMEM((CHUNK,), jnp.int32),
      pltpu.VMEM((CHUNK,), jnp.int32),
      pltpu.SemaphoreType.DMA,
      pltpu.SemaphoreType.DMA,
      pltpu.SemaphoreType.DMA,
      pltpu.SemaphoreType.DMA,
  ]

  def body(tab, src1d, dst1d, agg_out, agg_sh,
           rows, rowsb, srcb, srcbb, dstb, dstbb, sema, semb, semc, semd):
    cid = lax.axis_index("c")
    sid = lax.axis_index("s")

    def zrow(i, _):
      for k in range(D // L):
        rows[i, pl.ds(k * L, L)] = jnp.zeros((L,), jnp.float32)
      return 0
    lax.fori_loop(0, CHUNK, zrow, 0)

    base_r = sid * ROWS_PER_TILE
    def zcp(i, _):
      pltpu.sync_copy(rows.at[pl.ds(0, ZROWS)],
                      agg_sh.at[pl.ds(base_r + i * ZROWS, ZROWS)])
      return 0
    lax.fori_loop(0, ROWS_PER_TILE // ZROWS, zcp, 0)
    plsc.subcore_barrier()

    if feat_split or with_deg:
      ebase = sid * (EPAD // NS)
      nchunks = EPAD // NS // CHUNK
    else:
      wid = cid * NS + sid
      ebase = wid * (EPAD // (NC * NS))
      nchunks = EPAD // (NC * NS) // CHUNK

    def start_gather(e0, sb, rb, sm):
      pltpu.sync_copy(src1d.at[pl.ds(e0, CHUNK)], sb)
      if feat_split:
        off = cid * NPAD
        for k in range(CHUNK // L):
          v = sb[pl.ds(k * L, L)]
          sb[pl.ds(k * L, L)] = v + off
      return pltpu.async_copy(tab.at[sb], rb, sm)

    npairs = nchunks // 2

    def pair(p_, _):
      eA = ebase + (2 * p_) * CHUNK
      eB = eA + CHUNK
      pltpu.sync_copy(dst1d.at[pl.ds(eA, CHUNK)], dstb)
      pltpu.sync_copy(dst1d.at[pl.ds(eB, CHUNK)], dstbb)
      pltpu.make_async_copy(tab.at[srcb], rows, sema).wait()
      sA = pltpu.async_copy(rows, agg_sh.at[dstb], semc, add=True)
      pltpu.make_async_copy(tab.at[srcbb], rowsb, semb).wait()
      sB = pltpu.async_copy(rowsb, agg_sh.at[dstbb], semd, add=True)
      sA.wait()
      sB.wait()
      @pl.when(p_ + 1 < npairs)
      def _():
        start_gather(eA + 2 * CHUNK, srcb, rows, sema)
        start_gather(eB + 2 * CHUNK, srcbb, rowsb, semb)
      return 0

    def pair_deg(p_, _):
      eA = ebase + (2 * p_) * CHUNK
      eB = eA + CHUNK
      pltpu.sync_copy(dst1d.at[pl.ds(eA, CHUNK)], dstb)
      pltpu.sync_copy(dst1d.at[pl.ds(eB, CHUNK)], dstbb)
      sA = pltpu.async_copy(rows, agg_sh.at[dstb], semc, add=True)
      sB = pltpu.async_copy(rowsb, agg_sh.at[dstbb], semd, add=True)
      sA.wait()
      sB.wait()
      return 0

    if with_deg:
      @pl.when(cid == 0)
      def _():
        start_gather(ebase, srcb, rows, sema)
        start_gather(ebase + CHUNK, srcbb, rowsb, semb)
        lax.fori_loop(0, npairs, pair, 0)
      @pl.when(cid == 1)
      def _():
        def orow(i, _):
          for k in range(D // L):
            rows[i, pl.ds(k * L, L)] = jnp.full((L,), 1.0, jnp.float32)
            rowsb[i, pl.ds(k * L, L)] = jnp.full((L,), 1.0, jnp.float32)
          return 0
        lax.fori_loop(0, CHUNK, orow, 0)
        lax.fori_loop(0, npairs, pair_deg, 0)
    else:
      start_gather(ebase, srcb, rows, sema)
      start_gather(ebase + CHUNK, srcbb, rowsb, semb)
      lax.fori_loop(0, npairs, pair, 0)
    plsc.subcore_barrier()

    def cpo(i, _):
      rr = base_r + i * ZROWS
      pltpu.sync_copy(agg_sh.at[pl.ds(rr, ZROWS)], agg_out.at[cid, pl.ds(rr, ZROWS)])
      return 0
    lax.fori_loop(0, ROWS_PER_TILE // ZROWS, cpo, 0)

  return pl.kernel(body, out_type=out_type, mesh=mesh, scratch_types=scratch)


def _dot(a, b):
  return jnp.dot(a, b, preferred_element_type=jnp.float32)


def _t0_body(aggp_ref, x_ref, wl_ref, wr_ref, bl_ref,
             y_ref, rdeg_ref, s_ref, ss_ref):
  rdeg = 1.0 / jnp.maximum(aggp_ref[1], 1.0)
  rdeg_ref[...] = rdeg
  mean = aggp_ref[0] * rdeg
  y = _dot(mean, wl_ref[...]) + _dot(x_ref[...], wr_ref[...]) + bl_ref[...]
  y_ref[...] = y
  i = pl.program_id(0)
  rows = lax.broadcasted_iota(jnp.int32, (B, 1), 0) + i * B
  ym = jnp.where(rows < N, y, 0.0)
  s = jnp.sum(ym, axis=0, keepdims=True)
  ss = jnp.sum(ym * ym, axis=0, keepdims=True)
  @pl.when(i == 0)
  def _():
    s_ref[...] = s
    ss_ref[...] = ss
  @pl.when(i != 0)
  def _():
    s_ref[...] += s
    ss_ref[...] += ss


def _bn_coeffs(s_ref, ss_ref, g_ref, b_ref):
  m = s_ref[...] / N
  v = ss_ref[...] / N - m * m
  scale = g_ref[...] * lax.rsqrt(v + EPS)
  shift = b_ref[...] - m * scale
  return scale, shift


def _t0b_body(y_ref, s_ref, ss_ref, g_ref, b_ref, h2_ref):
  scale, shift = _bn_coeffs(s_ref, ss_ref, g_ref, b_ref)
  h = jnp.maximum(y_ref[...] * scale + shift, 0.0)
  h2_ref[0] = h[:, :D]
  h2_ref[1] = h[:, D:]


def _t1_body(agg_ref, h2_ref, rdeg_ref, wl_ref, wr_ref, bl_ref,
             y_ref, s_ref, ss_ref):
  rdeg = rdeg_ref[...]
  y = (_dot(agg_ref[0] * rdeg, wl_ref[0:D]) +
       _dot(agg_ref[1] * rdeg, wl_ref[D:]) +
       _dot(h2_ref[0], wr_ref[0:D]) +
       _dot(h2_ref[1], wr_ref[D:]) + bl_ref[...])
  y_ref[...] = y
  i = pl.program_id(0)
  rows = lax.broadcasted_iota(jnp.int32, (B, 1), 0) + i * B
  ym = jnp.where(rows < N, y, 0.0)
  s = jnp.sum(ym, axis=0, keepdims=True)
  ss = jnp.sum(ym * ym, axis=0, keepdims=True)
  @pl.when(i == 0)
  def _():
    s_ref[...] = s
    ss_ref[...] = ss
  @pl.when(i != 0)
  def _():
    s_ref[...] += s
    ss_ref[...] += ss


def _t1b_body(y_ref, s_ref, ss_ref, g_ref, b_ref, wl2_ref, h_ref, p_ref):
  scale, shift = _bn_coeffs(s_ref, ss_ref, g_ref, b_ref)
  h = jnp.maximum(y_ref[...] * scale + shift, 0.0)
  h_ref[...] = h
  p_ref[...] = _dot(h, wl2_ref[...])


def _t2_body(aggp_ref, rdeg_ref, h_ref, wr2_ref, bl2_ref, o_ref):
  mean_wl = (aggp_ref[0] + aggp_ref[1]) * rdeg_ref[...]
  o_ref[...] = mean_wl + _dot(h_ref[...], wr2_ref[...]) + bl2_ref[...]


def _row_spec(w):
  return pl.BlockSpec((B, w), lambda i: (i, 0))


def _full_spec(shape):
  return pl.BlockSpec(shape, lambda i: tuple(0 for _ in shape))


def _stacked_spec(w):
  return pl.BlockSpec((2, B, w), lambda i: (0, i, 0))


_t0_call = pl.pallas_call(
    _t0_body, grid=(G,),
    in_specs=[_stacked_spec(D), _row_spec(D),
              _full_spec((D, H2)), _full_spec((D, H2)), _full_spec((1, H2))],
    out_specs=[_row_spec(H2), _row_spec(D), _full_spec((1, H2)),
               _full_spec((1, H2))],
    out_shape=[jax.ShapeDtypeStruct((NPAD, H2), jnp.float32),
               jax.ShapeDtypeStruct((NPAD, D), jnp.float32),
               jax.ShapeDtypeStruct((1, H2), jnp.float32),
               jax.ShapeDtypeStruct((1, H2), jnp.float32)],
)

_t0b_call = pl.pallas_call(
    _t0b_body, grid=(G,),
    in_specs=[_row_spec(H2), _full_spec((1, H2)), _full_spec((1, H2)),
              _full_spec((1, H2)), _full_spec((1, H2))],
    out_specs=[_stacked_spec(D)],
    out_shape=[jax.ShapeDtypeStruct((2, NPAD, D), jnp.float32)],
)

_t1_call = pl.pallas_call(
    _t1_body, grid=(G,),
    in_specs=[_stacked_spec(D), _stacked_spec(D), _row_spec(D),
              _full_spec((H2, H2)), _full_spec((H2, H2)), _full_spec((1, H2))],
    out_specs=[_row_spec(H2), _full_spec((1, H2)), _full_spec((1, H2))],
    out_shape=[jax.ShapeDtypeStruct((NPAD, H2), jnp.float32),
               jax.ShapeDtypeStruct((1, H2), jnp.float32),
               jax.ShapeDtypeStruct((1, H2), jnp.float32)],
)

_t1b_call = pl.pallas_call(
    _t1b_body, grid=(G,),
    in_specs=[_row_spec(H2), _full_spec((1, H2)), _full_spec((1, H2)),
              _full_spec((1, H2)), _full_spec((1, H2)), _full_spec((H2, D))],
    out_specs=[_row_spec(H2), _row_spec(D)],
    out_shape=[jax.ShapeDtypeStruct((NPAD, H2), jnp.float32),
               jax.ShapeDtypeStruct((NPAD, D), jnp.float32)],
)

_t2_call = pl.pallas_call(
    _t2_body, grid=(G,),
    in_specs=[_stacked_spec(D), _row_spec(D), _row_spec(H2),
              _full_spec((H2, D)), _full_spec((1, D))],
    out_specs=[_row_spec(D)],
    out_shape=[jax.ShapeDtypeStruct((NPAD, D), jnp.float32)],
)

_sc_cache = {}


def _sc_agg(feat_split, with_deg):
  key = (feat_split, with_deg)
  if key not in _sc_cache:
    _sc_cache[key] = _make_sc_agg(feat_split, with_deg)
  return _sc_cache[key]


def kernel(x, edge_index, Wl0, bl0, Wr0, g0, b0,
           Wl1, bl1, Wr1, g1, b1, Wl2, bl2, Wr2):
  x_pad = jnp.zeros((NPAD, D), jnp.float32).at[:N].set(x)
  npe = EPAD - E
  pad_idx = N + (jnp.arange(npe, dtype=jnp.int32) % (NPAD - N))
  src1d = jnp.concatenate([edge_index[0], pad_idx])
  dst1d = jnp.concatenate([edge_index[1], pad_idx])

  (agg0,) = _sc_agg(False, True)(x_pad, src1d, dst1d)


  y0, rdeg, s0, ss0 = _t0_call(agg0, x_pad, Wl0, Wr0, bl0.reshape(1, H2))
  (h0st,) = _t0b_call(y0, s0, ss0, g0.reshape(1, H2), b0.reshape(1, H2))
  (agg1,) = _sc_agg(True, False)(h0st.reshape(2 * NPAD, D), src1d, dst1d)
  y1, s1, ss1 = _t1_call(agg1, h0st, rdeg, Wl1, Wr1, bl1.reshape(1, H2))
  h1, p = _t1b_call(y1, s1, ss1, g1.reshape(1, H2), b1.reshape(1, H2), Wl2)
  (agg2,) = _sc_agg(False, False)(p, src1d, dst1d)
  (out,) = _t2_call(agg2, rdeg, h1, Wr2, bl2.reshape(1, D))
  return out[:N]
